# Initial kernel scaffold; baseline (speedup 1.0000x reference)
import jax
import jax.numpy as jnp
from jax.experimental import pallas as pl


def kernel(x, edge_index, W, att):
    raise NotImplementedError("write your pallas kernel here")


# speedup vs baseline: 67.1295x; 67.1295x over previous
import functools

import jax
import jax.numpy as jnp
from jax import lax
from jax.experimental import pallas as pl
from jax.experimental.pallas import tpu as pltpu
from jax.experimental.pallas import tpu_sc as plsc

N = 10000
E = 320000
D = 128
HEADS = 4
HC = 32
NEG = 0.2
NP = 10008
NW = 32
NC = 2
EPW = E // NW
C1 = 2000
C2 = 400
CB = 80

f32 = jnp.float32
i32 = jnp.int32


def _iota16():
    return lax.broadcasted_iota(i32, (16,), 0)


def _tc1_body(x_ref, w_ref, adst_w_ref, asrc_w_ref, h_ref, apack_ref,
              apack_t_ref, bound_ref):
    h = jnp.dot(x_ref[...], w_ref[...], preferred_element_type=f32)
    h_ref[...] = h
    adst = jnp.dot(h, adst_w_ref[...], preferred_element_type=f32)
    asrc = jnp.dot(h, asrc_w_ref[...], preferred_element_type=f32)
    bound4 = jnp.max(adst, axis=0) + jnp.max(asrc, axis=0)
    bound_ref[...] = jnp.concatenate([bound4] * 4, axis=0)
    apack = jnp.concatenate([asrc, adst], axis=1)
    apack_p = jnp.concatenate(
        [apack, jnp.zeros((NP - N, 2 * HEADS), f32)], axis=0)
    apack_ref[...] = apack_p
    apack_t_ref[...] = jnp.transpose(apack_p, (1, 0))


def _tc1(x, W, adst_w, asrc_w):
    return pl.pallas_call(
        _tc1_body,
        out_shape=[
            jax.ShapeDtypeStruct((N, D), f32),
            jax.ShapeDtypeStruct((NP, 2 * HEADS), f32),
            jax.ShapeDtypeStruct((2 * HEADS, NP), f32),
            jax.ShapeDtypeStruct((16,), f32),
        ],
    )(x, W, adst_w, asrc_w)


def _sc_a1(row, col):
    mesh = plsc.VectorSubcoreMesh(core_axis_name="c", subcore_axis_name="s")

    @functools.partial(
        pl.kernel,
        out_type=jax.ShapeDtypeStruct((NW, NP), f32),
        mesh=mesh,
        compiler_params=pltpu.CompilerParams(needs_layout_passes=False),
        scratch_types=[
            pltpu.VMEM((C1,), i32),
            pltpu.VMEM((C1,), i32),
            pltpu.VMEM((NP,), f32),
        ],
    )
    def kern(row_hbm, col_hbm, dp_out, row_v, col_v, deg_l):
        cid = lax.axis_index("c")
        sid = lax.axis_index("s")
        wid = sid * NC + cid
        iota = _iota16()
        zeros = jnp.zeros((16,), f32)
        ones = jnp.ones((16,), f32)

        def zero_body(i, c):
            deg_l[pl.ds(i * 16, 16)] = zeros
            return c
        lax.fori_loop(0, NP // 16, zero_body, 0)

        def chunk_body(i, c):
            base = wid * EPW + i * C1
            pltpu.sync_copy(row_hbm.at[pl.ds(base, C1)], row_v)
            pltpu.sync_copy(col_hbm.at[pl.ds(base, C1)], col_v)

            def vec_body(b, c2):
                r = row_v[pl.ds(b * 16, 16)]
                cc = col_v[pl.ds(b * 16, 16)]
                rm = jnp.where(r == cc, N, r)
                for k in range(16):
                    plsc.addupdate_scatter(deg_l, [rm], ones,
                                           mask=iota == k)
                return c2
            lax.fori_loop(0, C1 // 16, vec_body, 0)
            return c
        lax.fori_loop(0, EPW // C1, chunk_body, 0)

        pltpu.sync_copy(deg_l, dp_out.at[wid])

    return kern(row, col)


def _sc_a2(row, col, apack_flat, bound16):
    mesh = plsc.VectorSubcoreMesh(core_axis_name="c", subcore_axis_name="s")

    @functools.partial(
        pl.kernel,
        out_type=[
            jax.ShapeDtypeStruct((E * HEADS,), f32),
            jax.ShapeDtypeStruct((NW, NP * HEADS), f32),
            jax.ShapeDtypeStruct((E,), i32),
            jax.ShapeDtypeStruct((E,), i32),
        ],
        mesh=mesh,
        compiler_params=pltpu.CompilerParams(needs_layout_passes=False),
        scratch_types=[
            pltpu.VMEM((C2,), i32),
            pltpu.VMEM((C2,), i32),
            pltpu.VMEM((C2,), i32),
            pltpu.VMEM((C2,), i32),
            pltpu.VMEM((C2 * HEADS,), f32),
            pltpu.VMEM((16,), f32),
            pltpu.VMEM((NP * 2 * HEADS,), f32),
            pltpu.VMEM((NP * HEADS,), f32),
        ],
    )
    def kern(row_hbm, col_hbm, apack_hbm, bound_hbm, p_out, sp_out,
             rowm_out, colm_out,
             row_v, col_v, rowm_v, colm_v, p_v, bnd_v, apack_l, s_l):
        cid = lax.axis_index("c")
        sid = lax.axis_index("s")
        wid = sid * NC + cid
        iota = _iota16()
        lane_e = iota // HEADS
        lane_h = iota % HEADS
        zeros = jnp.zeros((16,), f32)

        pltpu.sync_copy(apack_hbm, apack_l)
        pltpu.sync_copy(bound_hbm, bnd_v)
        bnd = bnd_v[...]

        def zero_body(i, c):
            s_l[pl.ds(i * 16, 16)] = zeros
            return c
        lax.fori_loop(0, NP * HEADS // 16, zero_body, 0)

        def chunk_body(i, c):
            base = wid * EPW + i * C2
            pltpu.sync_copy(row_hbm.at[pl.ds(base, C2)], row_v)
            pltpu.sync_copy(col_hbm.at[pl.ds(base, C2)], col_v)

            def mask_body(b, c2):
                r = row_v[pl.ds(b * 16, 16)]
                cc = col_v[pl.ds(b * 16, 16)]
                m = r == cc
                rowm_v[pl.ds(b * 16, 16)] = jnp.where(m, N, r)
                colm_v[pl.ds(b * 16, 16)] = jnp.where(m, N, cc)
                return c2
            lax.fori_loop(0, C2 // 16, mask_body, 0)

            def grp_body(a, c2):
                rrep = plsc.load_gather(rowm_v, [a * 4 + lane_e])
                crep = plsc.load_gather(colm_v, [a * 4 + lane_e])
                ga = plsc.load_gather(apack_l, [rrep * 8 + lane_h])
                gb = plsc.load_gather(apack_l, [crep * 8 + 4 + lane_h])
                alpha = ga + gb
                alpha = jnp.maximum(alpha, NEG * alpha)
                pv = jnp.exp(alpha - bnd)
                p_v[pl.ds(a * 16, 16)] = pv
                sidx = lane_h * NP + crep
                for ee in range(4):
                    plsc.addupdate_scatter(s_l, [sidx], pv,
                                           mask=lane_e == ee)
                return c2
            lax.fori_loop(0, C2 // 4, grp_body, 0)

            pltpu.sync_copy(p_v, p_out.at[pl.ds(base * HEADS, C2 * HEADS)])
            pltpu.sync_copy(rowm_v, rowm_out.at[pl.ds(base, C2)])
            pltpu.sync_copy(colm_v, colm_out.at[pl.ds(base, C2)])
            return c
        lax.fori_loop(0, EPW // C2, chunk_body, 0)

        pltpu.sync_copy(s_l, sp_out.at[wid])

    return kern(row, col, apack_flat, bound16)


def _tc3_body(h_ref, apack_t_ref, bound_ref, sp_ref, dp_ref,
              u_ref, hs_ref, oself_ref):
    deg = jnp.sum(dp_ref[...], axis=0, keepdims=True)
    iota = lax.broadcasted_iota(i32, (1, NP), 1)
    deg = deg + jnp.where(iota < N, 1.0, 0.0)
    dis = jnp.where(deg > 0, lax.rsqrt(deg), 0.0)
    asrc = apack_t_ref[:HEADS, :]
    adst = apack_t_ref[HEADS:, :]
    alphal = asrc + adst
    alphal = jnp.maximum(alphal, NEG * alphal)
    bound4 = jnp.reshape(bound_ref[...][:HEADS], (HEADS, 1))
    p_self = jnp.exp(alphal - bound4)
    S = jnp.sum(sp_ref[...], axis=0) + p_self
    u = dis / (S + 1e-16)
    u_ref[...] = u
    h_pad = jnp.concatenate([h_ref[...], jnp.zeros((NP - N, D), f32)], 0)
    dis_n = jnp.transpose(dis, (1, 0))
    hs = dis_n * h_pad
    hs_ref[...] = hs
    wl = jnp.transpose((p_self * u)[:, :N], (1, 0))
    wexp = jnp.reshape(
        jnp.broadcast_to(jnp.reshape(wl, (N, HEADS, 1)), (N, HEADS, HC)),
        (N, D))
    oself_ref[...] = wexp * hs[:N]


def _tc3(h, apack_t, bound16, sp, dp):
    return pl.pallas_call(
        _tc3_body,
        out_shape=[
            jax.ShapeDtypeStruct((HEADS, NP), f32),
            jax.ShapeDtypeStruct((NP, D), f32),
            jax.ShapeDtypeStruct((N, D), f32),
        ],
    )(h, apack_t, bound16, sp, dp)


def _sc_a3(colm, p_flat, u_flat):
    mesh = plsc.VectorSubcoreMesh(core_axis_name="c", subcore_axis_name="s")

    @functools.partial(
        pl.kernel,
        out_type=jax.ShapeDtypeStruct((E * HEADS,), f32),
        mesh=mesh,
        compiler_params=pltpu.CompilerParams(needs_layout_passes=False),
        scratch_types=[
            pltpu.VMEM((C2,), i32),
            pltpu.VMEM((C2 * HEADS,), f32),
            pltpu.VMEM((NP * HEADS,), f32),
        ],
    )
    def kern(colm_hbm, p_hbm, u_hbm, w_out, colm_v, p_v, u_l):
        cid = lax.axis_index("c")
        sid = lax.axis_index("s")
        wid = sid * NC + cid
        iota = _iota16()
        lane_e = iota // HEADS
        lane_h = iota % HEADS

        pltpu.sync_copy(u_hbm, u_l)

        def chunk_body(i, c):
            base = wid * EPW + i * C2
            pltpu.sync_copy(colm_hbm.at[pl.ds(base, C2)], colm_v)
            pltpu.sync_copy(p_hbm.at[pl.ds(base * HEADS, C2 * HEADS)], p_v)

            def grp_body(a, c2):
                crep = plsc.load_gather(colm_v, [a * 4 + lane_e])
                ue = plsc.load_gather(u_l, [lane_h * NP + crep])
                pv = p_v[pl.ds(a * 16, 16)]
                p_v[pl.ds(a * 16, 16)] = pv * ue
                return c2
            lax.fori_loop(0, C2 // 4, grp_body, 0)

            pltpu.sync_copy(p_v, w_out.at[pl.ds(base * HEADS, C2 * HEADS)])
            return c
        lax.fori_loop(0, EPW // C2, chunk_body, 0)

    return kern(colm, p_flat, u_flat)


def _sc_b(rowm, colm, w_flat, hs, z128):
    mesh = plsc.VectorSubcoreMesh(core_axis_name="c", subcore_axis_name="s")

    @functools.partial(
        pl.kernel,
        out_type=jax.ShapeDtypeStruct((NC, NP, D), f32),
        mesh=mesh,
        compiler_params=pltpu.CompilerParams(needs_layout_passes=False),
        scratch_types=[
            pltpu.VMEM((CB,), i32),
            pltpu.VMEM((CB,), i32),
            pltpu.VMEM((CB * HEADS,), f32),
            pltpu.VMEM((CB, D), f32),
            pltpu.VMEM_SHARED((NP, D), f32),
            pltpu.SemaphoreType.DMA,
        ],
    )
    def kern(rowm_hbm, colm_hbm, w_hbm, hs_hbm, z_hbm, o_out,
             rowm_v, colm_v, w_v, rows_v, O_sh, sem):
        cid = lax.axis_index("c")
        sid = lax.axis_index("s")
        wid = sid * NC + cid

        @pl.when(sid == 0)
        def _():
            pltpu.sync_copy(z_hbm, O_sh)

        plsc.subcore_barrier()

        gd = lax.GatherDimensionNumbers(
            offset_dims=(), collapsed_slice_dims=(0,), start_index_map=(0,))

        def chunk_body(i, c):
            base = wid * EPW + i * CB
            pltpu.sync_copy(rowm_hbm.at[pl.ds(base, CB)], rowm_v)
            pltpu.sync_copy(colm_hbm.at[pl.ds(base, CB)], colm_v)
            cp = pltpu.async_copy(hs_hbm.at[rowm_v], rows_v, sem)
            pltpu.sync_copy(w_hbm.at[pl.ds(base * HEADS, CB * HEADS)], w_v)
            cp.wait()

            def grp_body(a, c2):
                w = w_v[pl.ds(a * 16, 16)]
                for ee in range(4):
                    for h in range(HEADS):
                        wb = lax.gather(
                            w, jnp.full((16, 1), 4 * ee + h, i32), gd, (1,),
                            mode=lax.GatherScatterMode.PROMISE_IN_BOUNDS)
                        for j2 in range(2):
                            sl = pl.ds((2 * h + j2) * 16, 16)
                            rows_v[a * 4 + ee, sl] = rows_v[a * 4 + ee, sl] * wb
                return c2
            lax.fori_loop(0, CB // 4, grp_body, 0)

            pltpu.sync_copy(rows_v, O_sh.at[colm_v], add=True)
            return c
        lax.fori_loop(0, EPW // CB, chunk_body, 0)

        plsc.subcore_barrier()

        @pl.when(sid == 0)
        def _():
            pltpu.sync_copy(O_sh, o_out.at[cid])

    return kern(rowm, colm, w_flat, hs, z128)


def _tc5_body(op_ref, oself_ref, out_ref):
    out_ref[...] = op_ref[0, :N, :] + op_ref[1, :N, :] + oself_ref[...]


def _tc5(op, oself):
    return pl.pallas_call(
        _tc5_body,
        out_shape=jax.ShapeDtypeStruct((N, D), f32),
    )(op, oself)


def kernel(x, edge_index, W, att):
    row = edge_index[0]
    col = edge_index[1]
    att_dst = att[0, :, :HC]
    att_src = att[0, :, HC:]
    eye = jnp.eye(HEADS, dtype=f32)
    adst_w = jnp.einsum("hc,hk->hck", att_dst, eye).reshape(D, HEADS)
    asrc_w = jnp.einsum("hc,hk->hck", att_src, eye).reshape(D, HEADS)
    z128 = jnp.zeros((NP, D), f32)

    h, apack, apack_t, bound16 = _tc1(x, W, adst_w, asrc_w)
    dp = _sc_a1(row, col)
    p_flat, sp, rowm, colm = _sc_a2(row, col, apack.reshape(-1), bound16)
    u, hs, oself = _tc3(h, apack_t, bound16,
                        sp.reshape(NW, HEADS, NP), dp)
    w_flat = _sc_a3(colm, p_flat, u.reshape(-1))
    op = _sc_b(rowm, colm, w_flat, hs, z128)
    return _tc5(op, oself)


# speedup vs baseline: 85.9873x; 1.2809x over previous
import functools

import jax
import jax.numpy as jnp
from jax import lax
from jax.experimental import pallas as pl
from jax.experimental.pallas import tpu as pltpu
from jax.experimental.pallas import tpu_sc as plsc

N = 10000
E = 320000
D = 128
HEADS = 4
HC = 32
NEG = 0.2
NP = 10008
NW = 32
NC = 2
EPW = E // NW
C1 = 2000
C2 = 400
CB = 80

f32 = jnp.float32
i32 = jnp.int32


def _iota16():
    return lax.broadcasted_iota(i32, (16,), 0)


def _tc1_body(x_ref, w_ref, adst_w_ref, asrc_w_ref, h_ref, apack_ref,
              apack_t_ref, bound_ref):
    h = jnp.dot(x_ref[...], w_ref[...], preferred_element_type=f32)
    h_ref[...] = h
    adst = jnp.dot(h, adst_w_ref[...], preferred_element_type=f32)
    asrc = jnp.dot(h, asrc_w_ref[...], preferred_element_type=f32)
    bound4 = jnp.max(adst, axis=0) + jnp.max(asrc, axis=0)
    bound_ref[...] = jnp.concatenate([bound4] * 4, axis=0)
    apack = jnp.concatenate([asrc, adst], axis=1)
    apack_p = jnp.concatenate(
        [apack, jnp.zeros((NP - N, 2 * HEADS), f32)], axis=0)
    apack_ref[...] = apack_p
    apack_t_ref[...] = jnp.transpose(apack_p, (1, 0))


def _tc1(x, W, adst_w, asrc_w):
    return pl.pallas_call(
        _tc1_body,
        out_shape=[
            jax.ShapeDtypeStruct((N, D), f32),
            jax.ShapeDtypeStruct((NP, 2 * HEADS), f32),
            jax.ShapeDtypeStruct((2 * HEADS, NP), f32),
            jax.ShapeDtypeStruct((16,), f32),
        ],
    )(x, W, adst_w, asrc_w)


def _sc_a1(row, col):
    mesh = plsc.VectorSubcoreMesh(core_axis_name="c", subcore_axis_name="s")

    @functools.partial(
        pl.kernel,
        out_type=jax.ShapeDtypeStruct((NW, NP), f32),
        mesh=mesh,
        compiler_params=pltpu.CompilerParams(needs_layout_passes=False),
        scratch_types=[
            pltpu.VMEM((C1,), i32),
            pltpu.VMEM((C1,), i32),
            pltpu.VMEM((NP,), f32),
        ],
    )
    def kern(row_hbm, col_hbm, dp_out, row_v, col_v, deg_l):
        cid = lax.axis_index("c")
        sid = lax.axis_index("s")
        wid = sid * NC + cid
        iota = _iota16()
        zeros = jnp.zeros((16,), f32)
        ones = jnp.ones((16,), f32)

        def zero_body(i, c):
            deg_l[pl.ds(i * 16, 16)] = zeros
            return c
        lax.fori_loop(0, NP // 16, zero_body, 0)

        def chunk_body(i, c):
            base = wid * EPW + i * C1
            pltpu.sync_copy(row_hbm.at[pl.ds(base, C1)], row_v)
            pltpu.sync_copy(col_hbm.at[pl.ds(base, C1)], col_v)

            def vec_body(b, c2):
                r = row_v[pl.ds(b * 16, 16)]
                cc = col_v[pl.ds(b * 16, 16)]
                rm = jnp.where(r == cc, N, r)
                for k in range(16):
                    plsc.addupdate_scatter(deg_l, [rm], ones,
                                           mask=iota == k)
                return c2
            lax.fori_loop(0, C1 // 16, vec_body, 0)
            return c
        lax.fori_loop(0, EPW // C1, chunk_body, 0)

        pltpu.sync_copy(deg_l, dp_out.at[wid])

    return kern(row, col)


def _sc_a2(row, col, apack_flat, bound16):
    mesh = plsc.VectorSubcoreMesh(core_axis_name="c", subcore_axis_name="s")

    @functools.partial(
        pl.kernel,
        out_type=[
            jax.ShapeDtypeStruct((E * HEADS,), f32),
            jax.ShapeDtypeStruct((NW, NP * HEADS), f32),
            jax.ShapeDtypeStruct((E,), i32),
            jax.ShapeDtypeStruct((E,), i32),
        ],
        mesh=mesh,
        compiler_params=pltpu.CompilerParams(needs_layout_passes=False),
        scratch_types=[
            pltpu.VMEM((C2,), i32),
            pltpu.VMEM((C2,), i32),
            pltpu.VMEM((C2,), i32),
            pltpu.VMEM((C2,), i32),
            pltpu.VMEM((C2 * HEADS,), f32),
            pltpu.VMEM((16,), f32),
            pltpu.VMEM((NP * 2 * HEADS,), f32),
            pltpu.VMEM((NP * HEADS,), f32),
        ],
    )
    def kern(row_hbm, col_hbm, apack_hbm, bound_hbm, p_out, sp_out,
             rowm_out, colm_out,
             row_v, col_v, rowm_v, colm_v, p_v, bnd_v, apack_l, s_l):
        cid = lax.axis_index("c")
        sid = lax.axis_index("s")
        wid = sid * NC + cid
        iota = _iota16()
        lane_e = iota // HEADS
        lane_h = iota % HEADS
        zeros = jnp.zeros((16,), f32)

        pltpu.sync_copy(apack_hbm, apack_l)
        pltpu.sync_copy(bound_hbm, bnd_v)
        bnd = bnd_v[...]

        def zero_body(i, c):
            s_l[pl.ds(i * 16, 16)] = zeros
            return c
        lax.fori_loop(0, NP * HEADS // 16, zero_body, 0)

        def chunk_body(i, c):
            base = wid * EPW + i * C2
            pltpu.sync_copy(row_hbm.at[pl.ds(base, C2)], row_v)
            pltpu.sync_copy(col_hbm.at[pl.ds(base, C2)], col_v)

            def mask_body(b, c2):
                r = row_v[pl.ds(b * 16, 16)]
                cc = col_v[pl.ds(b * 16, 16)]
                m = r == cc
                rowm_v[pl.ds(b * 16, 16)] = jnp.where(m, N, r)
                colm_v[pl.ds(b * 16, 16)] = jnp.where(m, N, cc)
                return c2
            lax.fori_loop(0, C2 // 16, mask_body, 0)

            def grp_body(a, c2):
                rrep = plsc.load_gather(rowm_v, [a * 4 + lane_e])
                crep = plsc.load_gather(colm_v, [a * 4 + lane_e])
                ga = plsc.load_gather(apack_l, [rrep * 8 + lane_h])
                gb = plsc.load_gather(apack_l, [crep * 8 + 4 + lane_h])
                alpha = ga + gb
                alpha = jnp.maximum(alpha, NEG * alpha)
                pv = jnp.exp(alpha - bnd)
                p_v[pl.ds(a * 16, 16)] = pv
                sidx = lane_h * NP + crep
                for ee in range(4):
                    plsc.addupdate_scatter(s_l, [sidx], pv,
                                           mask=lane_e == ee)
                return c2
            lax.fori_loop(0, C2 // 4, grp_body, 0)

            pltpu.sync_copy(p_v, p_out.at[pl.ds(base * HEADS, C2 * HEADS)])
            pltpu.sync_copy(rowm_v, rowm_out.at[pl.ds(base, C2)])
            pltpu.sync_copy(colm_v, colm_out.at[pl.ds(base, C2)])
            return c
        lax.fori_loop(0, EPW // C2, chunk_body, 0)

        pltpu.sync_copy(s_l, sp_out.at[wid])

    return kern(row, col, apack_flat, bound16)


def _tc3_body(h_ref, apack_t_ref, bound_ref, sp_ref, dp_ref,
              u_ref, hs_ref, oself_ref):
    deg = jnp.sum(dp_ref[...], axis=0, keepdims=True)
    iota = lax.broadcasted_iota(i32, (1, NP), 1)
    deg = deg + jnp.where(iota < N, 1.0, 0.0)
    dis = jnp.where(deg > 0, lax.rsqrt(deg), 0.0)
    asrc = apack_t_ref[:HEADS, :]
    adst = apack_t_ref[HEADS:, :]
    alphal = asrc + adst
    alphal = jnp.maximum(alphal, NEG * alphal)
    bound4 = jnp.reshape(bound_ref[...][:HEADS], (HEADS, 1))
    p_self = jnp.exp(alphal - bound4)
    S = jnp.sum(sp_ref[...], axis=0) + p_self
    u = dis / (S + 1e-16)
    u_ref[...] = u
    h_pad = jnp.concatenate([h_ref[...], jnp.zeros((NP - N, D), f32)], 0)
    dis_n = jnp.transpose(dis, (1, 0))
    hs = dis_n * h_pad
    hs_ref[...] = hs
    wl = jnp.transpose((p_self * u)[:, :N], (1, 0))
    wexp = jnp.reshape(
        jnp.broadcast_to(jnp.reshape(wl, (N, HEADS, 1)), (N, HEADS, HC)),
        (N, D))
    oself_ref[...] = wexp * hs[:N]


def _tc3(h, apack_t, bound16, sp, dp):
    return pl.pallas_call(
        _tc3_body,
        out_shape=[
            jax.ShapeDtypeStruct((HEADS, NP), f32),
            jax.ShapeDtypeStruct((NP, D), f32),
            jax.ShapeDtypeStruct((N, D), f32),
        ],
    )(h, apack_t, bound16, sp, dp)


def _sc_a3(colm, p_flat, u_flat):
    mesh = plsc.VectorSubcoreMesh(core_axis_name="c", subcore_axis_name="s")

    @functools.partial(
        pl.kernel,
        out_type=jax.ShapeDtypeStruct((E * HEADS,), f32),
        mesh=mesh,
        compiler_params=pltpu.CompilerParams(needs_layout_passes=False),
        scratch_types=[
            pltpu.VMEM((C2,), i32),
            pltpu.VMEM((C2 * HEADS,), f32),
            pltpu.VMEM((NP * HEADS,), f32),
        ],
    )
    def kern(colm_hbm, p_hbm, u_hbm, w_out, colm_v, p_v, u_l):
        cid = lax.axis_index("c")
        sid = lax.axis_index("s")
        wid = sid * NC + cid
        iota = _iota16()
        lane_e = iota // HEADS
        lane_h = iota % HEADS

        pltpu.sync_copy(u_hbm, u_l)

        def chunk_body(i, c):
            base = wid * EPW + i * C2
            pltpu.sync_copy(colm_hbm.at[pl.ds(base, C2)], colm_v)
            pltpu.sync_copy(p_hbm.at[pl.ds(base * HEADS, C2 * HEADS)], p_v)

            def grp_body(a, c2):
                crep = plsc.load_gather(colm_v, [a * 4 + lane_e])
                ue = plsc.load_gather(u_l, [lane_h * NP + crep])
                pv = p_v[pl.ds(a * 16, 16)]
                p_v[pl.ds(a * 16, 16)] = pv * ue
                return c2
            lax.fori_loop(0, C2 // 4, grp_body, 0)

            pltpu.sync_copy(p_v, w_out.at[pl.ds(base * HEADS, C2 * HEADS)])
            return c
        lax.fori_loop(0, EPW // C2, chunk_body, 0)

    return kern(colm, p_flat, u_flat)


SB = 2000
NSB = EPW // SB
MID = 200
NMID = SB // MID
SUB = 50
SPM = MID // SUB


def _sc_b(rowm2, colm2, w_flat, hs, z128):
    mesh = plsc.VectorSubcoreMesh(core_axis_name="c", subcore_axis_name="s")

    @functools.partial(
        pl.kernel,
        out_type=jax.ShapeDtypeStruct((NC, NP, D), f32),
        mesh=mesh,
        compiler_params=pltpu.CompilerParams(needs_layout_passes=False),
        scratch_types=[
            pltpu.VMEM((SB // SUB, SUB), i32),
            pltpu.VMEM((SB // SUB, SUB), i32),
            pltpu.VMEM((SB * HEADS,), f32),
            pltpu.VMEM((MID, D), f32),
            pltpu.VMEM_SHARED((NP, D), f32),
            pltpu.SemaphoreType.DMA,
            pltpu.SemaphoreType.DMA,
            pltpu.SemaphoreType.DMA,
        ],
    )
    def kern(rowm_hbm, colm_hbm, w_hbm, hs_hbm, z_hbm, o_out,
             rowm_s, colm_s, w_s, rows_v, O_sh, sem_ld, sem_g, sem_s):
        cid = lax.axis_index("c")
        sid = lax.axis_index("s")
        wid = sid * NC + cid

        @pl.when(sid == 0)
        def _():
            pltpu.sync_copy(z_hbm, O_sh)

        plsc.subcore_barrier()

        gd = lax.GatherDimensionNumbers(
            offset_dims=(), collapsed_slice_dims=(0,), start_index_map=(0,))

        def super_body(s, c):
            ebase = wid * EPW + s * SB
            rbase = pl.multiple_of((wid * EPW + s * SB) // SUB, 8)
            c1 = pltpu.async_copy(
                rowm_hbm.at[pl.ds(rbase, SB // SUB)], rowm_s, sem_ld)
            c2 = pltpu.async_copy(
                colm_hbm.at[pl.ds(rbase, SB // SUB)], colm_s, sem_ld)
            c3 = pltpu.async_copy(
                w_hbm.at[pl.ds(ebase * HEADS, SB * HEADS)], w_s, sem_ld)
            c1.wait()
            c2.wait()
            c3.wait()

            def mid_body(m, c2_):
                gs = [pltpu.async_copy(
                    hs_hbm.at[rowm_s.at[SPM * m + j]],
                    rows_v.at[pl.ds(j * SUB, SUB)], sem_g)
                    for j in range(SPM)]
                for g in gs:
                    g.wait()

                def grp_body(a, c3_):
                    woff = (m * MID + a * 4) * HEADS
                    w = w_s[pl.ds(woff, 16)]
                    for ee in range(4):
                        for h in range(HEADS):
                            wb = lax.gather(
                                w, jnp.full((16, 1), 4 * ee + h, i32),
                                gd, (1,),
                                mode=lax.GatherScatterMode.PROMISE_IN_BOUNDS)
                            for j2 in range(2):
                                sl = pl.ds((2 * h + j2) * 16, 16)
                                rows_v[a * 4 + ee, sl] = (
                                    rows_v[a * 4 + ee, sl] * wb)
                    return c3_
                lax.fori_loop(0, MID // 4, grp_body, 0)

                ss = [pltpu.async_copy(
                    rows_v.at[pl.ds(j * SUB, SUB)],
                    O_sh.at[colm_s.at[SPM * m + j]], sem_s, add=True)
                    for j in range(SPM)]
                for s_ in ss:
                    s_.wait()
                return c2_
            lax.fori_loop(0, NMID, mid_body, 0)
            return c
        lax.fori_loop(0, NSB, super_body, 0)

        plsc.subcore_barrier()

        @pl.when(sid == 0)
        def _():
            pltpu.sync_copy(O_sh, o_out.at[cid])

    return kern(rowm2, colm2, w_flat, hs, z128)


def _tc5_body(op_ref, oself_ref, out_ref):
    out_ref[...] = op_ref[0, :N, :] + op_ref[1, :N, :] + oself_ref[...]


def _tc5(op, oself):
    return pl.pallas_call(
        _tc5_body,
        out_shape=jax.ShapeDtypeStruct((N, D), f32),
    )(op, oself)


def kernel(x, edge_index, W, att):
    row = edge_index[0]
    col = edge_index[1]
    att_dst = att[0, :, :HC]
    att_src = att[0, :, HC:]
    eye = jnp.eye(HEADS, dtype=f32)
    adst_w = jnp.einsum("hc,hk->hck", att_dst, eye).reshape(D, HEADS)
    asrc_w = jnp.einsum("hc,hk->hck", att_src, eye).reshape(D, HEADS)
    z128 = jnp.zeros((NP, D), f32)

    h, apack, apack_t, bound16 = _tc1(x, W, adst_w, asrc_w)
    dp = _sc_a1(row, col)
    p_flat, sp, rowm, colm = _sc_a2(row, col, apack.reshape(-1), bound16)
    u, hs, oself = _tc3(h, apack_t, bound16,
                        sp.reshape(NW, HEADS, NP), dp)
    w_flat = _sc_a3(colm, p_flat, u.reshape(-1))
    op = _sc_b(rowm.reshape(E // SUB, SUB), colm.reshape(E // SUB, SUB),
               w_flat, hs, z128)
    return _tc5(op, oself)


# speedup vs baseline: 97.7733x; 1.1371x over previous
import functools

import jax
import jax.numpy as jnp
from jax import lax
from jax.experimental import pallas as pl
from jax.experimental.pallas import tpu as pltpu
from jax.experimental.pallas import tpu_sc as plsc

N = 10000
E = 320000
D = 128
HEADS = 4
HC = 32
NEG = 0.2
NP = 10008
NW = 32
NC = 2
EPW = E // NW
C1 = 2000
C2 = 400
CB = 80

f32 = jnp.float32
i32 = jnp.int32


def _iota16():
    return lax.broadcasted_iota(i32, (16,), 0)


def _tc1_body(x_ref, w_ref, adst_w_ref, asrc_w_ref, h_ref, apack_ref,
              apack_t_ref, bound_ref):
    h = jnp.dot(x_ref[...], w_ref[...], preferred_element_type=f32)
    h_ref[...] = h
    adst = jnp.dot(h, adst_w_ref[...], preferred_element_type=f32)
    asrc = jnp.dot(h, asrc_w_ref[...], preferred_element_type=f32)
    bound4 = jnp.max(adst, axis=0) + jnp.max(asrc, axis=0)
    bound_ref[...] = jnp.concatenate([bound4] * 4, axis=0)
    apack = jnp.concatenate([asrc, adst], axis=1)
    apack_p = jnp.concatenate(
        [apack, jnp.zeros((NP - N, 2 * HEADS), f32)], axis=0)
    apack_ref[...] = apack_p
    apack_t_ref[...] = jnp.transpose(apack_p, (1, 0))


def _tc1(x, W, adst_w, asrc_w):
    return pl.pallas_call(
        _tc1_body,
        out_shape=[
            jax.ShapeDtypeStruct((N, D), f32),
            jax.ShapeDtypeStruct((NP, 2 * HEADS), f32),
            jax.ShapeDtypeStruct((2 * HEADS, NP), f32),
            jax.ShapeDtypeStruct((16,), f32),
        ],
    )(x, W, adst_w, asrc_w)


def _sc_a1(row, col):
    mesh = plsc.VectorSubcoreMesh(core_axis_name="c", subcore_axis_name="s")

    @functools.partial(
        pl.kernel,
        out_type=jax.ShapeDtypeStruct((NW, NP), f32),
        mesh=mesh,
        compiler_params=pltpu.CompilerParams(needs_layout_passes=False),
        scratch_types=[
            pltpu.VMEM((C1,), i32),
            pltpu.VMEM((C1,), i32),
            pltpu.VMEM((NP,), f32),
        ],
    )
    def kern(row_hbm, col_hbm, dp_out, row_v, col_v, deg_l):
        cid = lax.axis_index("c")
        sid = lax.axis_index("s")
        wid = sid * NC + cid
        iota = _iota16()
        zeros = jnp.zeros((16,), f32)
        ones = jnp.ones((16,), f32)

        def zero_body(i, c):
            deg_l[pl.ds(i * 16, 16)] = zeros
            return c
        lax.fori_loop(0, NP // 16, zero_body, 0)

        def chunk_body(i, c):
            base = wid * EPW + i * C1
            pltpu.sync_copy(row_hbm.at[pl.ds(base, C1)], row_v)
            pltpu.sync_copy(col_hbm.at[pl.ds(base, C1)], col_v)

            def vec_body(b, c2):
                r = row_v[pl.ds(b * 16, 16)]
                cc = col_v[pl.ds(b * 16, 16)]
                rm = jnp.where(r == cc, N, r)
                for k in range(16):
                    plsc.addupdate_scatter(deg_l, [rm], ones,
                                           mask=iota == k)
                return c2
            lax.fori_loop(0, C1 // 16, vec_body, 0)
            return c
        lax.fori_loop(0, EPW // C1, chunk_body, 0)

        pltpu.sync_copy(deg_l, dp_out.at[wid])

    return kern(row, col)


def _sc_a2(row, col, apack_flat, bound16):
    mesh = plsc.VectorSubcoreMesh(core_axis_name="c", subcore_axis_name="s")

    @functools.partial(
        pl.kernel,
        out_type=[
            jax.ShapeDtypeStruct((E * HEADS,), f32),
            jax.ShapeDtypeStruct((NW, NP * HEADS), f32),
            jax.ShapeDtypeStruct((E,), i32),
            jax.ShapeDtypeStruct((E,), i32),
        ],
        mesh=mesh,
        compiler_params=pltpu.CompilerParams(needs_layout_passes=False),
        scratch_types=[
            pltpu.VMEM((C2,), i32),
            pltpu.VMEM((C2,), i32),
            pltpu.VMEM((C2,), i32),
            pltpu.VMEM((C2,), i32),
            pltpu.VMEM((C2 * HEADS,), f32),
            pltpu.VMEM((16,), f32),
            pltpu.VMEM((NP * 2 * HEADS,), f32),
            pltpu.VMEM((NP * HEADS,), f32),
        ],
    )
    def kern(row_hbm, col_hbm, apack_hbm, bound_hbm, p_out, sp_out,
             rowm_out, colm_out,
             row_v, col_v, rowm_v, colm_v, p_v, bnd_v, apack_l, s_l):
        cid = lax.axis_index("c")
        sid = lax.axis_index("s")
        wid = sid * NC + cid
        iota = _iota16()
        lane_e = iota // HEADS
        lane_h = iota % HEADS
        zeros = jnp.zeros((16,), f32)

        pltpu.sync_copy(apack_hbm, apack_l)
        pltpu.sync_copy(bound_hbm, bnd_v)
        bnd = bnd_v[...]

        def zero_body(i, c):
            s_l[pl.ds(i * 16, 16)] = zeros
            return c
        lax.fori_loop(0, NP * HEADS // 16, zero_body, 0)

        def chunk_body(i, c):
            base = wid * EPW + i * C2
            pltpu.sync_copy(row_hbm.at[pl.ds(base, C2)], row_v)
            pltpu.sync_copy(col_hbm.at[pl.ds(base, C2)], col_v)

            def mask_body(b, c2):
                r = row_v[pl.ds(b * 16, 16)]
                cc = col_v[pl.ds(b * 16, 16)]
                m = r == cc
                rowm_v[pl.ds(b * 16, 16)] = jnp.where(m, N, r)
                colm_v[pl.ds(b * 16, 16)] = jnp.where(m, N, cc)
                return c2
            lax.fori_loop(0, C2 // 16, mask_body, 0)

            def grp_body(a, c2):
                rrep = plsc.load_gather(rowm_v, [a * 4 + lane_e])
                crep = plsc.load_gather(colm_v, [a * 4 + lane_e])
                ga = plsc.load_gather(apack_l, [rrep * 8 + lane_h])
                gb = plsc.load_gather(apack_l, [crep * 8 + 4 + lane_h])
                alpha = ga + gb
                alpha = jnp.maximum(alpha, NEG * alpha)
                pv = jnp.exp(alpha - bnd)
                p_v[pl.ds(a * 16, 16)] = pv
                sidx = lane_h * NP + crep
                for ee in range(4):
                    plsc.addupdate_scatter(s_l, [sidx], pv,
                                           mask=lane_e == ee)
                return c2
            lax.fori_loop(0, C2 // 4, grp_body, 0)

            pltpu.sync_copy(p_v, p_out.at[pl.ds(base * HEADS, C2 * HEADS)])
            pltpu.sync_copy(rowm_v, rowm_out.at[pl.ds(base, C2)])
            pltpu.sync_copy(colm_v, colm_out.at[pl.ds(base, C2)])
            return c
        lax.fori_loop(0, EPW // C2, chunk_body, 0)

        pltpu.sync_copy(s_l, sp_out.at[wid])

    return kern(row, col, apack_flat, bound16)


def _tc3_body(h_ref, apack_t_ref, bound_ref, sp_ref, dp_ref,
              u_ref, hs_ref, oself_ref):
    deg = jnp.sum(dp_ref[...], axis=0, keepdims=True)
    iota = lax.broadcasted_iota(i32, (1, NP), 1)
    deg = deg + jnp.where(iota < N, 1.0, 0.0)
    dis = jnp.where(deg > 0, lax.rsqrt(deg), 0.0)
    asrc = apack_t_ref[:HEADS, :]
    adst = apack_t_ref[HEADS:, :]
    alphal = asrc + adst
    alphal = jnp.maximum(alphal, NEG * alphal)
    bound4 = jnp.reshape(bound_ref[...][:HEADS], (HEADS, 1))
    p_self = jnp.exp(alphal - bound4)
    S = jnp.sum(sp_ref[...], axis=0) + p_self
    u = dis / (S + 1e-16)
    u_ref[...] = u
    h_pad = jnp.concatenate([h_ref[...], jnp.zeros((NP - N, D), f32)], 0)
    dis_n = jnp.transpose(dis, (1, 0))
    hs = dis_n * h_pad
    hs_ref[...] = hs
    wl = jnp.transpose((p_self * u)[:, :N], (1, 0))
    wexp = jnp.reshape(
        jnp.broadcast_to(jnp.reshape(wl, (N, HEADS, 1)), (N, HEADS, HC)),
        (N, D))
    oself_ref[...] = wexp * hs[:N]


def _tc3(h, apack_t, bound16, sp, dp):
    return pl.pallas_call(
        _tc3_body,
        out_shape=[
            jax.ShapeDtypeStruct((HEADS, NP), f32),
            jax.ShapeDtypeStruct((NP, D), f32),
            jax.ShapeDtypeStruct((N, D), f32),
        ],
    )(h, apack_t, bound16, sp, dp)


def _sc_a3(colm, p_flat, u_flat):
    mesh = plsc.VectorSubcoreMesh(core_axis_name="c", subcore_axis_name="s")

    @functools.partial(
        pl.kernel,
        out_type=jax.ShapeDtypeStruct((E * HEADS,), f32),
        mesh=mesh,
        compiler_params=pltpu.CompilerParams(needs_layout_passes=False),
        scratch_types=[
            pltpu.VMEM((C2,), i32),
            pltpu.VMEM((C2 * HEADS,), f32),
            pltpu.VMEM((NP * HEADS,), f32),
        ],
    )
    def kern(colm_hbm, p_hbm, u_hbm, w_out, colm_v, p_v, u_l):
        cid = lax.axis_index("c")
        sid = lax.axis_index("s")
        wid = sid * NC + cid
        iota = _iota16()
        lane_e = iota // HEADS
        lane_h = iota % HEADS

        pltpu.sync_copy(u_hbm, u_l)

        def chunk_body(i, c):
            base = wid * EPW + i * C2
            pltpu.sync_copy(colm_hbm.at[pl.ds(base, C2)], colm_v)
            pltpu.sync_copy(p_hbm.at[pl.ds(base * HEADS, C2 * HEADS)], p_v)

            def grp_body(a, c2):
                crep = plsc.load_gather(colm_v, [a * 4 + lane_e])
                ue = plsc.load_gather(u_l, [lane_h * NP + crep])
                pv = p_v[pl.ds(a * 16, 16)]
                p_v[pl.ds(a * 16, 16)] = pv * ue
                return c2
            lax.fori_loop(0, C2 // 4, grp_body, 0)

            pltpu.sync_copy(p_v, w_out.at[pl.ds(base * HEADS, C2 * HEADS)])
            return c
        lax.fori_loop(0, EPW // C2, chunk_body, 0)

    return kern(colm, p_flat, u_flat)


SB = 2000
NSB = EPW // SB
MID = 100
NMID = SB // MID
NPAIR = NMID // 2
SUB = 50
SPM = MID // SUB


def _sc_b(rowm2, colm2, w_flat, hs, z128):
    mesh = plsc.VectorSubcoreMesh(core_axis_name="c", subcore_axis_name="s")

    @functools.partial(
        pl.kernel,
        out_type=jax.ShapeDtypeStruct((NC, NP, D), f32),
        mesh=mesh,
        compiler_params=pltpu.CompilerParams(needs_layout_passes=False),
        scratch_types=[
            pltpu.VMEM((SB // SUB, SUB), i32),
            pltpu.VMEM((SB // SUB, SUB), i32),
            pltpu.VMEM((SB * HEADS,), f32),
            pltpu.VMEM((MID, D), f32),
            pltpu.VMEM((MID, D), f32),
            pltpu.VMEM_SHARED((NP, D), f32),
            pltpu.SemaphoreType.DMA,
            pltpu.SemaphoreType.DMA,
            pltpu.SemaphoreType.DMA,
            pltpu.SemaphoreType.DMA,
            pltpu.SemaphoreType.DMA,
        ],
    )
    def kern(rowm_hbm, colm_hbm, w_hbm, hs_hbm, z_hbm, o_out,
             rowm_s, colm_s, w_s, rows0, rows1, O_sh,
             sem_ld, sem_g0, sem_g1, sem_s0, sem_s1):
        cid = lax.axis_index("c")
        sid = lax.axis_index("s")
        wid = sid * NC + cid

        @pl.when(sid == 0)
        def _():
            pltpu.sync_copy(z_hbm, O_sh)

        plsc.subcore_barrier()

        gd = lax.GatherDimensionNumbers(
            offset_dims=(), collapsed_slice_dims=(0,), start_index_map=(0,))

        def issue_g(m, rows_ref, sem):
            for j in range(SPM):
                pltpu.async_copy(
                    hs_hbm.at[rowm_s.at[SPM * m + j]],
                    rows_ref.at[pl.ds(j * SUB, SUB)], sem)

        def drain_g(rows_ref, sem):
            for j in range(SPM):
                pltpu.make_async_copy(
                    hs_hbm.at[rowm_s.at[j]],
                    rows_ref.at[pl.ds(j * SUB, SUB)], sem).wait()

        def issue_s(m, rows_ref, sem):
            for j in range(SPM):
                pltpu.async_copy(
                    rows_ref.at[pl.ds(j * SUB, SUB)],
                    O_sh.at[colm_s.at[SPM * m + j]], sem, add=True)

        def drain_s(rows_ref, sem):
            for j in range(SPM):
                pltpu.make_async_copy(
                    rows_ref.at[pl.ds(j * SUB, SUB)],
                    O_sh.at[colm_s.at[j]], sem).wait()

        def compute(m, rows_ref):
            def grp_body(a, c_):
                woff = (m * MID + a * 4) * HEADS
                w = w_s[pl.ds(woff, 16)]
                for ee in range(4):
                    for h in range(HEADS):
                        wb = lax.gather(
                            w, jnp.full((16, 1), 4 * ee + h, i32),
                            gd, (1,),
                            mode=lax.GatherScatterMode.PROMISE_IN_BOUNDS)
                        for j2 in range(2):
                            sl = pl.ds((2 * h + j2) * 16, 16)
                            rows_ref[a * 4 + ee, sl] = (
                                rows_ref[a * 4 + ee, sl] * wb)
                return c_
            lax.fori_loop(0, MID // 4, grp_body, 0)

        def super_body(s, c):
            @pl.when(s > 0)
            def _():
                drain_s(rows1, sem_s1)

            ebase = wid * EPW + s * SB
            rbase = pl.multiple_of((wid * EPW + s * SB) // SUB, 8)
            c1 = pltpu.async_copy(
                rowm_hbm.at[pl.ds(rbase, SB // SUB)], rowm_s, sem_ld)
            c2 = pltpu.async_copy(
                colm_hbm.at[pl.ds(rbase, SB // SUB)], colm_s, sem_ld)
            c3 = pltpu.async_copy(
                w_hbm.at[pl.ds(ebase * HEADS, SB * HEADS)], w_s, sem_ld)
            c1.wait()
            c2.wait()
            c3.wait()

            issue_g(0, rows0, sem_g0)

            def pair_body(p, c2_):
                m0 = 2 * p
                m1 = 2 * p + 1

                @pl.when(p > 0)
                def _():
                    drain_s(rows1, sem_s1)
                issue_g(m1, rows1, sem_g1)
                drain_g(rows0, sem_g0)
                compute(m0, rows0)
                issue_s(m0, rows0, sem_s0)
                drain_g(rows1, sem_g1)
                compute(m1, rows1)
                drain_s(rows0, sem_s0)

                @pl.when(p < NPAIR - 1)
                def _():
                    issue_g(m0 + 2, rows0, sem_g0)
                issue_s(m1, rows1, sem_s1)
                return c2_
            lax.fori_loop(0, NPAIR, pair_body, 0)
            return c
        lax.fori_loop(0, NSB, super_body, 0)

        drain_s(rows1, sem_s1)
        plsc.subcore_barrier()

        @pl.when(sid == 0)
        def _():
            pltpu.sync_copy(O_sh, o_out.at[cid])

    return kern(rowm2, colm2, w_flat, hs, z128)


def _tc5_body(op_ref, oself_ref, out_ref):
    out_ref[...] = op_ref[0, :N, :] + op_ref[1, :N, :] + oself_ref[...]


def _tc5(op, oself):
    return pl.pallas_call(
        _tc5_body,
        out_shape=jax.ShapeDtypeStruct((N, D), f32),
    )(op, oself)


def kernel(x, edge_index, W, att):
    row = edge_index[0]
    col = edge_index[1]
    att_dst = att[0, :, :HC]
    att_src = att[0, :, HC:]
    eye = jnp.eye(HEADS, dtype=f32)
    adst_w = jnp.einsum("hc,hk->hck", att_dst, eye).reshape(D, HEADS)
    asrc_w = jnp.einsum("hc,hk->hck", att_src, eye).reshape(D, HEADS)
    z128 = jnp.zeros((NP, D), f32)

    h, apack, apack_t, bound16 = _tc1(x, W, adst_w, asrc_w)
    dp = _sc_a1(row, col)
    p_flat, sp, rowm, colm = _sc_a2(row, col, apack.reshape(-1), bound16)
    u, hs, oself = _tc3(h, apack_t, bound16,
                        sp.reshape(NW, HEADS, NP), dp)
    w_flat = _sc_a3(colm, p_flat, u.reshape(-1))
    op = _sc_b(rowm.reshape(E // SUB, SUB), colm.reshape(E // SUB, SUB),
               w_flat, hs, z128)
    return _tc5(op, oself)


# speedup vs baseline: 104.5474x; 1.0693x over previous
import functools

import jax
import jax.numpy as jnp
from jax import lax
from jax.experimental import pallas as pl
from jax.experimental.pallas import tpu as pltpu
from jax.experimental.pallas import tpu_sc as plsc

N = 10000
E = 320000
D = 128
HEADS = 4
HC = 32
NEG = 0.2
NP = 10008
NW = 32
NC = 2
EPW = E // NW
C1 = 2000
C2 = 400
CB = 80

f32 = jnp.float32
i32 = jnp.int32


def _iota16():
    return lax.broadcasted_iota(i32, (16,), 0)


def _tc1_body(x_ref, w_ref, adst_w_ref, asrc_w_ref, h_ref, apack_ref,
              apack_t_ref, bound_ref):
    h = jnp.dot(x_ref[...], w_ref[...], preferred_element_type=f32)
    h_ref[...] = h
    adst = jnp.dot(h, adst_w_ref[...], preferred_element_type=f32)
    asrc = jnp.dot(h, asrc_w_ref[...], preferred_element_type=f32)
    bound4 = jnp.max(adst, axis=0) + jnp.max(asrc, axis=0)
    bound_ref[...] = jnp.concatenate([bound4] * 4, axis=0)
    apack = jnp.concatenate([asrc, adst], axis=1)
    apack_p = jnp.concatenate(
        [apack, jnp.zeros((NP - N, 2 * HEADS), f32)], axis=0)
    apack_ref[...] = apack_p
    apack_t_ref[...] = jnp.transpose(apack_p, (1, 0))


def _tc1(x, W, adst_w, asrc_w):
    return pl.pallas_call(
        _tc1_body,
        out_shape=[
            jax.ShapeDtypeStruct((N, D), f32),
            jax.ShapeDtypeStruct((NP, 2 * HEADS), f32),
            jax.ShapeDtypeStruct((2 * HEADS, NP), f32),
            jax.ShapeDtypeStruct((16,), f32),
        ],
    )(x, W, adst_w, asrc_w)


def _sc_a1(row, col):
    mesh = plsc.VectorSubcoreMesh(core_axis_name="c", subcore_axis_name="s")

    @functools.partial(
        pl.kernel,
        out_type=jax.ShapeDtypeStruct((NW, NP), f32),
        mesh=mesh,
        compiler_params=pltpu.CompilerParams(needs_layout_passes=False),
        scratch_types=[
            pltpu.VMEM((C1,), i32),
            pltpu.VMEM((C1,), i32),
            pltpu.VMEM((NP,), f32),
        ],
    )
    def kern(row_hbm, col_hbm, dp_out, row_v, col_v, deg_l):
        cid = lax.axis_index("c")
        sid = lax.axis_index("s")
        wid = sid * NC + cid
        iota = _iota16()
        zeros = jnp.zeros((16,), f32)
        ones = jnp.ones((16,), f32)

        def zero_body(i, c):
            deg_l[pl.ds(i * 16, 16)] = zeros
            return c
        lax.fori_loop(0, NP // 16, zero_body, 0)

        def chunk_body(i, c):
            base = wid * EPW + i * C1
            pltpu.sync_copy(row_hbm.at[pl.ds(base, C1)], row_v)
            pltpu.sync_copy(col_hbm.at[pl.ds(base, C1)], col_v)

            def vec_body(b, c2):
                r = row_v[pl.ds(b * 16, 16)]
                cc = col_v[pl.ds(b * 16, 16)]
                rm = jnp.where(r == cc, N, r)
                for k in range(16):
                    plsc.addupdate_scatter(deg_l, [rm], ones,
                                           mask=iota == k)
                return c2
            lax.fori_loop(0, C1 // 16, vec_body, 0)
            return c
        lax.fori_loop(0, EPW // C1, chunk_body, 0)

        pltpu.sync_copy(deg_l, dp_out.at[wid])

    return kern(row, col)


def _sc_a2(row, col, apack_flat, bound16):
    mesh = plsc.VectorSubcoreMesh(core_axis_name="c", subcore_axis_name="s")

    @functools.partial(
        pl.kernel,
        out_type=[
            jax.ShapeDtypeStruct((E * HEADS,), f32),
            jax.ShapeDtypeStruct((NW, NP * HEADS), f32),
            jax.ShapeDtypeStruct((E,), i32),
            jax.ShapeDtypeStruct((E,), i32),
        ],
        mesh=mesh,
        compiler_params=pltpu.CompilerParams(needs_layout_passes=False),
        scratch_types=[
            pltpu.VMEM((C2,), i32),
            pltpu.VMEM((C2,), i32),
            pltpu.VMEM((C2,), i32),
            pltpu.VMEM((C2,), i32),
            pltpu.VMEM((C2 * HEADS,), f32),
            pltpu.VMEM((16,), f32),
            pltpu.VMEM((NP * 2 * HEADS,), f32),
            pltpu.VMEM((NP * HEADS,), f32),
            pltpu.SemaphoreType.DMA,
            pltpu.SemaphoreType.DMA,
        ],
    )
    def kern(row_hbm, col_hbm, apack_hbm, bound_hbm, p_out, sp_out,
             rowm_out, colm_out,
             row_v, col_v, rowm_v, colm_v, p_v, bnd_v, apack_l, s_l,
             sem_ld, sem_st):
        cid = lax.axis_index("c")
        sid = lax.axis_index("s")
        wid = sid * NC + cid
        iota = _iota16()
        lane_e = iota // HEADS
        lane_h = iota % HEADS
        zeros = jnp.zeros((16,), f32)

        pltpu.sync_copy(apack_hbm, apack_l)
        pltpu.sync_copy(bound_hbm, bnd_v)
        bnd = bnd_v[...]

        def zero_body(i, c):
            s_l[pl.ds(i * 16, 16)] = zeros
            return c
        lax.fori_loop(0, NP * HEADS // 16, zero_body, 0)

        def drain_st(base):
            pltpu.make_async_copy(
                p_v, p_out.at[pl.ds(base * HEADS, C2 * HEADS)],
                sem_st).wait()
            pltpu.make_async_copy(
                rowm_v, rowm_out.at[pl.ds(base, C2)], sem_st).wait()
            pltpu.make_async_copy(
                colm_v, colm_out.at[pl.ds(base, C2)], sem_st).wait()

        def chunk_body(i, c):
            base = wid * EPW + i * C2
            l1 = pltpu.async_copy(row_hbm.at[pl.ds(base, C2)], row_v, sem_ld)
            l2 = pltpu.async_copy(col_hbm.at[pl.ds(base, C2)], col_v, sem_ld)

            @pl.when(i > 0)
            def _():
                drain_st(base)
            l1.wait()
            l2.wait()

            def mask_body(b, c2):
                r = row_v[pl.ds(b * 16, 16)]
                cc = col_v[pl.ds(b * 16, 16)]
                m = r == cc
                rowm_v[pl.ds(b * 16, 16)] = jnp.where(m, N, r)
                colm_v[pl.ds(b * 16, 16)] = jnp.where(m, N, cc)
                return c2
            lax.fori_loop(0, C2 // 16, mask_body, 0)

            def grp_body(a, c2):
                rrep = plsc.load_gather(rowm_v, [a * 4 + lane_e])
                crep = plsc.load_gather(colm_v, [a * 4 + lane_e])
                ga = plsc.load_gather(apack_l, [rrep * 8 + lane_h])
                gb = plsc.load_gather(apack_l, [crep * 8 + 4 + lane_h])
                alpha = ga + gb
                alpha = jnp.maximum(alpha, NEG * alpha)
                pv = jnp.exp(alpha - bnd)
                p_v[pl.ds(a * 16, 16)] = pv
                sidx = lane_h * NP + crep
                for ee in range(4):
                    plsc.addupdate_scatter(s_l, [sidx], pv,
                                           mask=lane_e == ee)
                return c2
            lax.fori_loop(0, C2 // 4, grp_body, 0)

            pltpu.async_copy(
                p_v, p_out.at[pl.ds(base * HEADS, C2 * HEADS)], sem_st)
            pltpu.async_copy(rowm_v, rowm_out.at[pl.ds(base, C2)], sem_st)
            pltpu.async_copy(colm_v, colm_out.at[pl.ds(base, C2)], sem_st)
            return c
        lax.fori_loop(0, EPW // C2, chunk_body, 0)

        drain_st(wid * EPW)
        pltpu.sync_copy(s_l, sp_out.at[wid])

    return kern(row, col, apack_flat, bound16)


def _tc3_body(h_ref, apack_t_ref, bound_ref, sp_ref, dp_ref,
              u_ref, hs_ref, oself_ref):
    deg = jnp.sum(dp_ref[...], axis=0, keepdims=True)
    iota = lax.broadcasted_iota(i32, (1, NP), 1)
    deg = deg + jnp.where(iota < N, 1.0, 0.0)
    dis = jnp.where(deg > 0, lax.rsqrt(deg), 0.0)
    asrc = apack_t_ref[:HEADS, :]
    adst = apack_t_ref[HEADS:, :]
    alphal = asrc + adst
    alphal = jnp.maximum(alphal, NEG * alphal)
    bound4 = jnp.reshape(bound_ref[...][:HEADS], (HEADS, 1))
    p_self = jnp.exp(alphal - bound4)
    S = jnp.sum(sp_ref[...], axis=0) + p_self
    u = dis / (S + 1e-16)
    u_ref[...] = u
    h_pad = jnp.concatenate([h_ref[...], jnp.zeros((NP - N, D), f32)], 0)
    dis_n = jnp.transpose(dis, (1, 0))
    hs = dis_n * h_pad
    hs_ref[...] = hs
    wl = jnp.transpose((p_self * u)[:, :N], (1, 0))
    wexp = jnp.reshape(
        jnp.broadcast_to(jnp.reshape(wl, (N, HEADS, 1)), (N, HEADS, HC)),
        (N, D))
    oself_ref[...] = wexp * hs[:N]


def _tc3(h, apack_t, bound16, sp, dp):
    return pl.pallas_call(
        _tc3_body,
        out_shape=[
            jax.ShapeDtypeStruct((HEADS, NP), f32),
            jax.ShapeDtypeStruct((NP, D), f32),
            jax.ShapeDtypeStruct((N, D), f32),
        ],
    )(h, apack_t, bound16, sp, dp)


def _sc_a3(colm, p_flat, u_flat):
    mesh = plsc.VectorSubcoreMesh(core_axis_name="c", subcore_axis_name="s")

    @functools.partial(
        pl.kernel,
        out_type=jax.ShapeDtypeStruct((E * HEADS,), f32),
        mesh=mesh,
        compiler_params=pltpu.CompilerParams(needs_layout_passes=False),
        scratch_types=[
            pltpu.VMEM((C2,), i32),
            pltpu.VMEM((C2 * HEADS,), f32),
            pltpu.VMEM((NP * HEADS,), f32),
            pltpu.SemaphoreType.DMA,
            pltpu.SemaphoreType.DMA,
        ],
    )
    def kern(colm_hbm, p_hbm, u_hbm, w_out, colm_v, p_v, u_l,
             sem_ld, sem_st):
        cid = lax.axis_index("c")
        sid = lax.axis_index("s")
        wid = sid * NC + cid
        iota = _iota16()
        lane_e = iota // HEADS
        lane_h = iota % HEADS

        pltpu.sync_copy(u_hbm, u_l)

        def drain_st(base):
            pltpu.make_async_copy(
                p_v, w_out.at[pl.ds(base * HEADS, C2 * HEADS)],
                sem_st).wait()

        def chunk_body(i, c):
            base = wid * EPW + i * C2
            l1 = pltpu.async_copy(
                colm_hbm.at[pl.ds(base, C2)], colm_v, sem_ld)

            @pl.when(i > 0)
            def _():
                drain_st(base)
            l2 = pltpu.async_copy(
                p_hbm.at[pl.ds(base * HEADS, C2 * HEADS)], p_v, sem_ld)
            l1.wait()
            l2.wait()

            def grp_body(a, c2):
                crep = plsc.load_gather(colm_v, [a * 4 + lane_e])
                ue = plsc.load_gather(u_l, [lane_h * NP + crep])
                pv = p_v[pl.ds(a * 16, 16)]
                p_v[pl.ds(a * 16, 16)] = pv * ue
                return c2
            lax.fori_loop(0, C2 // 4, grp_body, 0)

            pltpu.async_copy(
                p_v, w_out.at[pl.ds(base * HEADS, C2 * HEADS)], sem_st)
            return c
        lax.fori_loop(0, EPW // C2, chunk_body, 0)

        drain_st(wid * EPW)

    return kern(colm, p_flat, u_flat)


SB = 2000
NSB = EPW // SB
MID = 100
NMID = SB // MID
NPAIR = NMID // 2
SUB = 50
SPM = MID // SUB


def _sc_b(rowm2, colm2, w_flat, hs, z128):
    mesh = plsc.VectorSubcoreMesh(core_axis_name="c", subcore_axis_name="s")

    @functools.partial(
        pl.kernel,
        out_type=jax.ShapeDtypeStruct((NC, NP, D), f32),
        mesh=mesh,
        compiler_params=pltpu.CompilerParams(needs_layout_passes=False),
        scratch_types=[
            pltpu.VMEM((SB // SUB, SUB), i32),
            pltpu.VMEM((SB // SUB, SUB), i32),
            pltpu.VMEM((SB * HEADS,), f32),
            pltpu.VMEM((MID, D), f32),
            pltpu.VMEM((MID, D), f32),
            pltpu.VMEM_SHARED((NP, D), f32),
            pltpu.SemaphoreType.DMA,
            pltpu.SemaphoreType.DMA,
            pltpu.SemaphoreType.DMA,
            pltpu.SemaphoreType.DMA,
            pltpu.SemaphoreType.DMA,
        ],
    )
    def kern(rowm_hbm, colm_hbm, w_hbm, hs_hbm, z_hbm, o_out,
             rowm_s, colm_s, w_s, rows0, rows1, O_sh,
             sem_ld, sem_g0, sem_g1, sem_s0, sem_s1):
        cid = lax.axis_index("c")
        sid = lax.axis_index("s")
        wid = sid * NC + cid

        @pl.when(sid == 0)
        def _():
            pltpu.sync_copy(z_hbm, O_sh)

        plsc.subcore_barrier()

        gd = lax.GatherDimensionNumbers(
            offset_dims=(), collapsed_slice_dims=(0,), start_index_map=(0,))

        def issue_g(m, rows_ref, sem):
            for j in range(SPM):
                pltpu.async_copy(
                    hs_hbm.at[rowm_s.at[SPM * m + j]],
                    rows_ref.at[pl.ds(j * SUB, SUB)], sem)

        def drain_g(rows_ref, sem):
            for j in range(SPM):
                pltpu.make_async_copy(
                    hs_hbm.at[rowm_s.at[j]],
                    rows_ref.at[pl.ds(j * SUB, SUB)], sem).wait()

        def issue_s(m, rows_ref, sem):
            for j in range(SPM):
                pltpu.async_copy(
                    rows_ref.at[pl.ds(j * SUB, SUB)],
                    O_sh.at[colm_s.at[SPM * m + j]], sem, add=True)

        def drain_s(rows_ref, sem):
            for j in range(SPM):
                pltpu.make_async_copy(
                    rows_ref.at[pl.ds(j * SUB, SUB)],
                    O_sh.at[colm_s.at[j]], sem).wait()

        def compute(m, rows_ref):
            def grp_body(a, c_):
                woff = (m * MID + a * 4) * HEADS
                w = w_s[pl.ds(woff, 16)]
                for ee in range(4):
                    for h in range(HEADS):
                        wb = lax.gather(
                            w, jnp.full((16, 1), 4 * ee + h, i32),
                            gd, (1,),
                            mode=lax.GatherScatterMode.PROMISE_IN_BOUNDS)
                        for j2 in range(2):
                            sl = pl.ds((2 * h + j2) * 16, 16)
                            rows_ref[a * 4 + ee, sl] = (
                                rows_ref[a * 4 + ee, sl] * wb)
                return c_
            lax.fori_loop(0, MID // 4, grp_body, 0)

        def super_body(s, c):
            @pl.when(s > 0)
            def _():
                drain_s(rows1, sem_s1)

            ebase = wid * EPW + s * SB
            rbase = pl.multiple_of((wid * EPW + s * SB) // SUB, 8)
            c1 = pltpu.async_copy(
                rowm_hbm.at[pl.ds(rbase, SB // SUB)], rowm_s, sem_ld)
            c2 = pltpu.async_copy(
                colm_hbm.at[pl.ds(rbase, SB // SUB)], colm_s, sem_ld)
            c3 = pltpu.async_copy(
                w_hbm.at[pl.ds(ebase * HEADS, SB * HEADS)], w_s, sem_ld)
            c1.wait()
            c2.wait()
            c3.wait()

            issue_g(0, rows0, sem_g0)

            def pair_body(p, c2_):
                m0 = 2 * p
                m1 = 2 * p + 1

                @pl.when(p > 0)
                def _():
                    drain_s(rows1, sem_s1)
                issue_g(m1, rows1, sem_g1)
                drain_g(rows0, sem_g0)
                compute(m0, rows0)
                issue_s(m0, rows0, sem_s0)
                drain_g(rows1, sem_g1)
                compute(m1, rows1)
                drain_s(rows0, sem_s0)

                @pl.when(p < NPAIR - 1)
                def _():
                    issue_g(m0 + 2, rows0, sem_g0)
                issue_s(m1, rows1, sem_s1)
                return c2_
            lax.fori_loop(0, NPAIR, pair_body, 0)
            return c
        lax.fori_loop(0, NSB, super_body, 0)

        drain_s(rows1, sem_s1)
        plsc.subcore_barrier()

        @pl.when(sid == 0)
        def _():
            pltpu.sync_copy(O_sh, o_out.at[cid])

    return kern(rowm2, colm2, w_flat, hs, z128)


def _tc5_body(op_ref, oself_ref, out_ref):
    out_ref[...] = op_ref[0, :N, :] + op_ref[1, :N, :] + oself_ref[...]


def _tc5(op, oself):
    return pl.pallas_call(
        _tc5_body,
        out_shape=jax.ShapeDtypeStruct((N, D), f32),
    )(op, oself)


def kernel(x, edge_index, W, att):
    row = edge_index[0]
    col = edge_index[1]
    att_dst = att[0, :, :HC]
    att_src = att[0, :, HC:]
    eye = jnp.eye(HEADS, dtype=f32)
    adst_w = jnp.einsum("hc,hk->hck", att_dst, eye).reshape(D, HEADS)
    asrc_w = jnp.einsum("hc,hk->hck", att_src, eye).reshape(D, HEADS)
    z128 = jnp.zeros((NP, D), f32)

    h, apack, apack_t, bound16 = _tc1(x, W, adst_w, asrc_w)
    dp = _sc_a1(row, col)
    p_flat, sp, rowm, colm = _sc_a2(row, col, apack.reshape(-1), bound16)
    u, hs, oself = _tc3(h, apack_t, bound16,
                        sp.reshape(NW, HEADS, NP), dp)
    w_flat = _sc_a3(colm, p_flat, u.reshape(-1))
    op = _sc_b(rowm.reshape(E // SUB, SUB), colm.reshape(E // SUB, SUB),
               w_flat, hs, z128)
    return _tc5(op, oself)
